# baseline (speedup 1.0000x reference)
import functools

import jax
import jax.numpy as jnp
import numpy as np
from jax import lax
from jax.experimental import pallas as pl
from jax.experimental.pallas import tpu as pltpu
from jax.experimental.pallas import tpu_sc as plsc

NC, NS, LANES = 2, 16, 16
NW = NC * NS

GT, PCT, EQ, HASH = 25, 14, 10, 11
NEG = -1e9

_DTBL = np.zeros(32, np.int32)
_DTBL[6] = 1; _DTBL[8] = 1
_DTBL[7] = -1; _DTBL[9] = -1


@functools.lru_cache(maxsize=None)
def _build(B, L, V):
    assert B % (NW * LANES) == 0 and L % 4 == 0 and V == 32
    rows_w = B // NW
    RB = min(128, rows_w)
    assert rows_w % RB == 0
    nblk = rows_w // RB

    mesh = plsc.VectorSubcoreMesh(
        core_axis_name="c", subcore_axis_name="s",
        num_cores=NC, num_subcores=NS)

    @functools.partial(
        pl.kernel,
        out_type=jax.ShapeDtypeStruct((B, V), jnp.float32),
        mesh=mesh,
        compiler_params=pltpu.CompilerParams(needs_layout_passes=False),
        scratch_types=[
            pltpu.VMEM((RB, L), jnp.int32),
            pltpu.VMEM((RB, V), jnp.float32),
            pltpu.VMEM((32,), jnp.int32),
        ],
    )
    def sc_kernel(tok_hbm, log_hbm, dtbl_hbm, out_hbm, tok_v, out_v, dtbl_v):
        iota = lax.iota(jnp.int32, LANES)
        wid = lax.axis_index("s") * NC + lax.axis_index("c")
        pltpu.sync_copy(dtbl_hbm, dtbl_v)

        def pair_of_groups(rowv0, rowv1):
            full = lambda k: jnp.full((LANES,), k, jnp.int32)
            rows = (rowv0, rowv1)
            lastv = [plsc.load_gather(tok_v, [r, full(L - 1)]) for r in rows]
            t197 = [plsc.load_gather(tok_v, [r, full(L - 3)]) for r in rows]
            t198 = [plsc.load_gather(tok_v, [r, full(L - 2)]) for r in rows]

            def body(_, carry):
                colv, c0, p0, r0, c1, p1, r1 = carry
                cs, ps, rs = [c0, c1], [p0, p1], [r0, r1]
                for _u in range(4):
                    for k in (0, 1):
                        t = plsc.load_gather(tok_v, [rows[k], colv])
                        d = plsc.load_gather(dtbl_v, [t])
                        cs[k] = jnp.maximum(cs[k] + d, 0)
                        rs[k] = jnp.where(
                            (ps[k] == lastv[k]) & (t == PCT), 1, rs[k])
                        ps[k] = t
                    colv = colv + 1
                return colv, cs[0], ps[0], rs[0], cs[1], ps[1], rs[1]

            zero = jnp.zeros((LANES,), jnp.int32)
            out = lax.fori_loop(
                0, L // 4, body,
                (zero, zero, full(-1), zero, zero, full(-1), zero))
            cs = (out[1], out[4])
            rs = (out[3], out[6])

            neg = jnp.full((LANES,), NEG, jnp.float32)
            for k in (0, 1):
                bracket = cs[k] > 0
                lv = lastv[k]
                ring_hit = (rs[k] > 0) & (lv >= 15) & (lv <= 24)
                bond = (((t197[k] == EQ) | (t197[k] == HASH)).astype(jnp.int32)
                        + ((t198[k] == EQ) | (t198[k] == HASH)).astype(jnp.int32)
                        + ((lv == EQ) | (lv == HASH)).astype(jnp.int32))
                maxb = jnp.where(lv == 0, 4,
                                 jnp.where(lv == 1, 2,
                                           jnp.where(lv == 2, 3, 99)))
                val_hit = (lv <= 2) & (bond >= maxb)
                plsc.addupdate_scatter(out_v, [rows[k], full(GT)], neg,
                                       mask=bracket)
                plsc.addupdate_scatter(out_v, [rows[k], lv], neg,
                                       mask=ring_hit)
                plsc.addupdate_scatter(out_v, [rows[k], full(EQ)], neg,
                                       mask=val_hit)
                plsc.addupdate_scatter(out_v, [rows[k], full(HASH)], neg,
                                       mask=val_hit)

        for blk in range(nblk):
            base = wid * rows_w + blk * RB
            pltpu.sync_copy(tok_hbm.at[pl.ds(base, RB)], tok_v)
            pltpu.sync_copy(log_hbm.at[pl.ds(base, RB)], out_v)
            for g in range(0, RB // LANES, 2):
                pair_of_groups(g * LANES + iota, (g + 1) * LANES + iota)
            pltpu.sync_copy(out_v, out_hbm.at[pl.ds(base, RB)])

    return sc_kernel


def kernel(logits, previous_tokens, current_step):
    del current_step
    tok = previous_tokens.astype(jnp.int32)
    logits = logits.astype(jnp.float32)
    B, L = tok.shape
    V = logits.shape[1]
    return _build(B, L, V)(tok, logits, jnp.asarray(_DTBL))


# speedup vs baseline: 1.0842x; 1.0842x over previous
import functools

import jax
import jax.numpy as jnp
import numpy as np
from jax import lax
from jax.experimental import pallas as pl
from jax.experimental.pallas import tpu as pltpu
from jax.experimental.pallas import tpu_sc as plsc

NC, NS, LANES = 2, 16, 16
NW = NC * NS

GT, PCT, EQ, HASH = 25, 14, 10, 11
NEG = -1e9

_DTBL = np.zeros(32, np.int32)
_DTBL[6] = 1; _DTBL[8] = 1
_DTBL[7] = -1; _DTBL[9] = -1


@functools.lru_cache(maxsize=None)
def _build(B, L, V):
    assert B % (NW * LANES) == 0 and L % 4 == 0 and V == 32
    rows_w = B // NW
    RB = min(128, rows_w)
    assert rows_w % RB == 0
    nblk = rows_w // RB

    mesh = plsc.VectorSubcoreMesh(
        core_axis_name="c", subcore_axis_name="s",
        num_cores=NC, num_subcores=NS)

    @functools.partial(
        pl.kernel,
        out_type=jax.ShapeDtypeStruct((B, V), jnp.float32),
        mesh=mesh,
        compiler_params=pltpu.CompilerParams(needs_layout_passes=False),
        scratch_types=[
            pltpu.VMEM((RB, L), jnp.int32),
            pltpu.VMEM((RB, V), jnp.float32),
            pltpu.VMEM((32,), jnp.int32),
        ],
    )
    def sc_kernel(tok_hbm, log_hbm, dtbl_hbm, out_hbm, tok_v, out_v, dtbl_v):
        iota = lax.iota(jnp.int32, LANES)
        wid = lax.axis_index("s") * NC + lax.axis_index("c")
        pltpu.sync_copy(dtbl_hbm, dtbl_v)

        def group(rowv):
            full = lambda k: jnp.full((LANES,), k, jnp.int32)
            lastv = plsc.load_gather(tok_v, [rowv, full(L - 1)])
            t197 = plsc.load_gather(tok_v, [rowv, full(L - 3)])
            t198 = plsc.load_gather(tok_v, [rowv, full(L - 2)])

            def body(_, carry):
                colv, c, prev, ring = carry
                for _u in range(8):
                    t = plsc.load_gather(tok_v, [rowv, colv])
                    d = plsc.load_gather(dtbl_v, [t])
                    c = jnp.maximum(c + d, 0)
                    ring = jnp.where((prev == lastv) & (t == PCT), 1, ring)
                    prev = t
                    colv = colv + 1
                return colv, c, prev, ring

            zero = jnp.zeros((LANES,), jnp.int32)
            _, c, prev, ring = lax.fori_loop(
                0, L // 8, body, (zero, zero, full(-1), zero))

            bracket = c > 0
            ring_hit = (ring > 0) & (lastv >= 15) & (lastv <= 24)
            bond = (((t197 == EQ) | (t197 == HASH)).astype(jnp.int32)
                    + ((t198 == EQ) | (t198 == HASH)).astype(jnp.int32)
                    + ((lastv == EQ) | (lastv == HASH)).astype(jnp.int32))
            maxb = jnp.where(lastv == 0, 4,
                             jnp.where(lastv == 1, 2,
                                       jnp.where(lastv == 2, 3, 99)))
            val_hit = (lastv <= 2) & (bond >= maxb)

            neg = jnp.full((LANES,), NEG, jnp.float32)
            plsc.addupdate_scatter(out_v, [rowv, full(GT)], neg, mask=bracket)
            plsc.addupdate_scatter(out_v, [rowv, lastv], neg, mask=ring_hit)
            plsc.addupdate_scatter(out_v, [rowv, full(EQ)], neg, mask=val_hit)
            plsc.addupdate_scatter(out_v, [rowv, full(HASH)], neg, mask=val_hit)

        for blk in range(nblk):
            base = wid * rows_w + blk * RB
            pltpu.sync_copy(tok_hbm.at[pl.ds(base, RB)], tok_v)
            pltpu.sync_copy(log_hbm.at[pl.ds(base, RB)], out_v)
            for g in range(RB // LANES):
                group(g * LANES + iota)
            pltpu.sync_copy(out_v, out_hbm.at[pl.ds(base, RB)])

    return sc_kernel


def kernel(logits, previous_tokens, current_step):
    del current_step
    tok = previous_tokens.astype(jnp.int32)
    logits = logits.astype(jnp.float32)
    B, L = tok.shape
    V = logits.shape[1]
    return _build(B, L, V)(tok, logits, jnp.asarray(_DTBL))
